# baseline (speedup 1.0000x reference)
import functools

import jax
import jax.numpy as jnp
from jax import lax
from jax.experimental import pallas as pl
from jax.experimental.pallas import tpu as pltpu
from jax.experimental.pallas import tpu_sc as plsc

K = 100000
D = 16
B = 16384

_NW = 32
_BPW = B // _NW

_ROWS_PER_LANE_ROW = 128 // D
_B2 = B // _ROWS_PER_LANE_ROW
_TC_BLOCK = 256


def _make_gather_kernel():
  mesh = plsc.VectorSubcoreMesh(core_axis_name="c", subcore_axis_name="s")

  @functools.partial(
      pl.kernel,
      mesh=mesh,
      compiler_params=pltpu.CompilerParams(use_tc_tiling_on_sc=False),
      out_type=jax.ShapeDtypeStruct((B, D), jnp.float32),
      scratch_types=[
          pltpu.VMEM((_BPW,), jnp.int32),
          pltpu.VMEM((_BPW, D), jnp.float32),
          pltpu.SemaphoreType.DMA,
      ],
  )
  def k(a_hbm, m_hbm, g_hbm, idx_v, rows_v, sem):
    wid = lax.axis_index("s") * 2 + lax.axis_index("c")
    base = wid * _BPW
    pltpu.sync_copy(a_hbm.at[pl.ds(base, _BPW)], idx_v)
    pltpu.async_copy(m_hbm.at[idx_v], rows_v, sem).wait()
    pltpu.sync_copy(rows_v, g_hbm.at[pl.ds(base, _BPW)])

  return k


_gather = _make_gather_kernel()


def _dot_sigmoid_body(x_ref, g_ref, s_ref, o_ref):
  p = x_ref[...] * g_ref[...]
  z = jnp.dot(p, s_ref[...], preferred_element_type=jnp.float32)
  o_ref[...] = 1.0 / (1.0 + jnp.exp(-z))


_dot_sigmoid = pl.pallas_call(
    _dot_sigmoid_body,
    grid=(_B2 // _TC_BLOCK,),
    in_specs=[
        pl.BlockSpec((_TC_BLOCK, 128), lambda i: (i, 0)),
        pl.BlockSpec((_TC_BLOCK, 128), lambda i: (i, 0)),
        pl.BlockSpec((128, _ROWS_PER_LANE_ROW), lambda i: (0, 0)),
    ],
    out_specs=pl.BlockSpec((_TC_BLOCK, _ROWS_PER_LANE_ROW), lambda i: (i, 0)),
    out_shape=jax.ShapeDtypeStruct((_B2, _ROWS_PER_LANE_ROW), jnp.float32),
)


@jax.jit
def kernel(X, A, m):
  g = _gather(A.astype(jnp.int32), m)
  sel = (lax.broadcasted_iota(jnp.int32, (128, _ROWS_PER_LANE_ROW), 0) // D
         == lax.broadcasted_iota(jnp.int32, (128, _ROWS_PER_LANE_ROW), 1)
         ).astype(jnp.float32)
  out = _dot_sigmoid(X.reshape(_B2, 128), g.reshape(_B2, 128), sel)
  return out.reshape(B)


# speedup vs baseline: 1.0071x; 1.0071x over previous
import functools

import jax
import jax.numpy as jnp
from jax import lax
from jax.experimental import pallas as pl
from jax.experimental.pallas import tpu as pltpu
from jax.experimental.pallas import tpu_sc as plsc

K = 100000
D = 16
B = 16384

_NW = 32
_BPW = B // _NW
_SUB = 8
_G1 = B // _SUB
_TPW = _BPW // _SUB

_TC_ROWS = 2048
_TC_G = _TC_ROWS // _SUB


def _make_gather_kernel():
  mesh = plsc.VectorSubcoreMesh(core_axis_name="c", subcore_axis_name="s")

  @functools.partial(
      pl.kernel,
      mesh=mesh,
      compiler_params=pltpu.CompilerParams(use_tc_tiling_on_sc=False),
      out_type=jax.ShapeDtypeStruct((_G1, _SUB, 128), jnp.float32),
      scratch_types=[
          pltpu.VMEM((_BPW,), jnp.int32),
          pltpu.VMEM((_BPW, D), jnp.float32),
          pltpu.VMEM((_TPW, _SUB, 128), jnp.float32),
          pltpu.SemaphoreType.DMA,
      ],
  )
  def k(a_hbm, m_hbm, g_hbm, idx_v, rows_v, pack_v, sem):
    wid = lax.axis_index("s") * 2 + lax.axis_index("c")
    base = wid * _BPW
    pltpu.sync_copy(a_hbm.at[pl.ds(base, _BPW)], idx_v)
    pltpu.async_copy(m_hbm.at[idx_v], rows_v, sem).wait()

    def body(t, _):
      for j in range(_SUB):
        pack_v[t, j, pl.ds(0, D)] = rows_v[t * _SUB + j, :]
      return _

    lax.fori_loop(0, _TPW, body, 0)
    pltpu.sync_copy(pack_v, g_hbm.at[pl.ds(wid * _TPW, _TPW)])

  return k


_gather = _make_gather_kernel()


def _dot_sigmoid_body(x_ref, g_ref, o_ref):
  g = g_ref[...][:, :, :D].reshape(_TC_ROWS, D)
  p = x_ref[...] * g
  z = jnp.sum(p, axis=1)
  o_ref[...] = 1.0 / (1.0 + jnp.exp(-z))


_dot_sigmoid = pl.pallas_call(
    _dot_sigmoid_body,
    grid=(B // _TC_ROWS,),
    in_specs=[
        pl.BlockSpec((_TC_ROWS, D), lambda i: (i, 0)),
        pl.BlockSpec((_TC_G, _SUB, 128), lambda i: (i, 0, 0)),
    ],
    out_specs=pl.BlockSpec((_TC_ROWS,), lambda i: (i,)),
    out_shape=jax.ShapeDtypeStruct((B,), jnp.float32),
)


@jax.jit
def kernel(X, A, m):
  g3 = _gather(A.astype(jnp.int32), m)
  return _dot_sigmoid(X, g3)
